# baseline (speedup 1.0000x reference)
import jax
import jax.numpy as jnp
from jax.experimental import pallas as pl
from jax.experimental.pallas import tpu as pltpu

_PATCH = 32
_GRID = 10
_H2 = 16
_W2 = 16


def _body(x_ref, tab_ref, o_ref, pos_scr):
    ph = pl.program_id(0)
    b = pl.program_id(1)

    @pl.when(b == 0)
    def _():
        hi = (ph * _GRID) // _H2
        for pw in range(_W2):
            hj = (pw * _GRID) // _W2
            row = hi * _GRID + hj
            pos_scr[pw, :] = tab_ref[row, :]

    X = x_ref[0]
    y = X.reshape(3, _PATCH, _W2, _PATCH)
    y = y.transpose(2, 1, 3, 0)
    o_ref[0] = y.reshape(_W2, _PATCH * _PATCH * 3) + pos_scr[...]


def kernel(x, position_embeddings):
    b, c, h, w = x.shape
    table = position_embeddings.reshape(_GRID * _GRID, _PATCH * _PATCH * 3)
    out = pl.pallas_call(
        _body,
        grid=(_H2, b),
        in_specs=[
            pl.BlockSpec((1, c, _PATCH, w), lambda ph, bi: (bi, 0, ph, 0)),
            pl.BlockSpec((_GRID * _GRID, _PATCH * _PATCH * 3),
                         lambda ph, bi: (0, 0)),
        ],
        out_specs=pl.BlockSpec((1, _W2, _PATCH * _PATCH * 3),
                               lambda ph, bi: (bi, ph, 0)),
        out_shape=jax.ShapeDtypeStruct((b, _H2 * _W2, _PATCH * _PATCH * 3),
                                       x.dtype),
        scratch_shapes=[pltpu.VMEM((_W2, _PATCH * _PATCH * 3), x.dtype)],
    )(x, table)
    return out


# speedup vs baseline: 6.1864x; 6.1864x over previous
import functools

import jax
import jax.numpy as jnp
import numpy as np
from jax import lax
from jax.experimental import pallas as pl
from jax.experimental.pallas import tpu as pltpu
from jax.experimental.pallas import tpu_sc as plsc

_PATCH = 32
_GRID = 10
_H2 = 16
_W2 = 16
_E = _PATCH * _PATCH * 3
_WHALF = 256
_STRIP = 8 * _E


def _hash_rows():
    i = np.arange(_H2)
    j = np.arange(_W2)
    hi = np.floor(i.astype(np.float32) * _GRID / _H2).astype(np.int32)
    hj = np.floor(j.astype(np.float32) * _GRID / _W2).astype(np.int32)
    return (hi[:, None] * _GRID + hj[None, :]).reshape(-1)


def _sc_body(x_hbm, tab_hbm, idx_hbm, out_hbm,
             in_v, out_f, tmpl2d_v, tmpl_f, idx8_v, sem_in, sem_out, sem_t):
    cidx = lax.axis_index("c")
    sidx = lax.axis_index("s")
    wid = sidx * 2 + cidx
    ph = wid // 2
    half = wid % 2
    p0 = ph * _W2 + half * 8
    row0 = ph * _PATCH
    w0 = half * _WHALF

    pltpu.sync_copy(idx_hbm.at[pl.ds(p0, 8)], idx8_v)
    pltpu.async_copy(tab_hbm.at[idx8_v], tmpl2d_v, sem_t).wait()

    for pw in range(8):
        def flat_body(g, carry, pw=pw):
            tmpl_f[pl.ds(pw * _E + g * 16, 16)] = tmpl2d_v[pw, pl.ds(g * 16, 16)]
            return carry
        lax.fori_loop(0, _E // 16, flat_body, 0)

    iota3 = lax.iota(jnp.int32, 16) * 3

    def b_body(b, carry):
        pltpu.async_copy(
            x_hbm.at[b, :, pl.ds(row0, _PATCH), pl.ds(w0, _WHALF)],
            in_v, sem_in).wait()

        for c in range(3):
            def kh_body(kh, carry2, c=c):
                colb0 = kh * 96 + c
                for s in range(16):
                    pw = s // 2
                    kw0 = (s % 2) * 16
                    fidx = iota3 + (pw * _E + colb0 + kw0 * 3)
                    v = in_v[c, kh, pl.ds(s * 16, 16)]
                    p = plsc.load_gather(tmpl_f, [fidx])
                    plsc.store_scatter(out_f, [fidx], v + p)
                return carry2
            lax.fori_loop(0, _PATCH, kh_body, 0)

        pltpu.async_copy(out_f, out_hbm.at[b, pl.ds(p0 * _E, _STRIP)],
                         sem_out).wait()
        return carry

    lax.fori_loop(0, x_hbm.shape[0], b_body, 0)


def kernel(x, position_embeddings):
    b = x.shape[0]
    table = position_embeddings.reshape(_GRID * _GRID, _E)
    idx = jnp.asarray(_hash_rows())

    mesh = plsc.VectorSubcoreMesh(core_axis_name="c", subcore_axis_name="s")
    run = functools.partial(
        pl.kernel,
        out_type=jax.ShapeDtypeStruct((b, _H2 * _W2 * _E), x.dtype),
        mesh=mesh,
        compiler_params=pltpu.CompilerParams(needs_layout_passes=False),
        scratch_types=[
            pltpu.VMEM((3, _PATCH, _WHALF), jnp.float32),
            pltpu.VMEM((_STRIP,), jnp.float32),
            pltpu.VMEM((8, _E), jnp.float32),
            pltpu.VMEM((_STRIP,), jnp.float32),
            pltpu.VMEM((8,), jnp.int32),
            pltpu.SemaphoreType.DMA,
            pltpu.SemaphoreType.DMA,
            pltpu.SemaphoreType.DMA,
        ],
    )(_sc_body)
    out = run(x, table, idx)
    return out.reshape(b, _H2 * _W2, _E)


# speedup vs baseline: 9.5400x; 1.5421x over previous
import functools

import jax
import jax.numpy as jnp
import numpy as np
from jax import lax
from jax.experimental import pallas as pl
from jax.experimental.pallas import tpu as pltpu
from jax.experimental.pallas import tpu_sc as plsc

_PATCH = 32
_GRID = 10
_H2 = 16
_W2 = 16
_E = _PATCH * _PATCH * 3
_WHALF = 256


def _hash_rows():
    i = np.arange(_H2)
    j = np.arange(_W2)
    hi = np.floor(i.astype(np.float32) * _GRID / _H2).astype(np.int32)
    hj = np.floor(j.astype(np.float32) * _GRID / _W2).astype(np.int32)
    return (hi[:, None] * _GRID + hj[None, :]).reshape(-1)


def _sc_body(x_hbm, tab_hbm, idx_hbm, out_hbm,
             in0, in1, out0, out1, tmpl_v, idx8_v,
             sem_i0, sem_i1, sem_o0, sem_o1, sem_t):
    cidx = lax.axis_index("c")
    sidx = lax.axis_index("s")
    wid = sidx * 2 + cidx
    ph = wid // 2
    half = wid % 2
    p0 = ph * _W2 + half * 8
    row0 = ph * _PATCH
    w0 = half * _WHALF
    nb = x_hbm.shape[0]

    pltpu.sync_copy(idx_hbm.at[pl.ds(p0, 8)], idx8_v)
    pltpu.async_copy(tab_hbm.at[idx8_v], tmpl_v, sem_t).wait()

    iota3 = lax.iota(jnp.int32, 16) * 3
    rows = [jnp.full((16,), pw, jnp.int32) for pw in range(8)]

    def din(b, buf, sem):
        return pltpu.make_async_copy(
            x_hbm.at[b, :, pl.ds(row0, _PATCH), pl.ds(w0, _WHALF)], buf, sem)

    def dout(b, buf, sem):
        return pltpu.make_async_copy(buf, out_hbm.at[b, pl.ds(p0, 8)], sem)

    def compute(in_v, out_v):
        for c in range(3):
            def kh_body(kh, carry, c=c):
                col0 = iota3 + (kh * 96 + c)
                col1 = col0 + 48
                for s in range(16):
                    row = rows[s // 2]
                    col = col0 if s % 2 == 0 else col1
                    v = in_v[c, kh, pl.ds(s * 16, 16)]
                    p = plsc.load_gather(tmpl_v, [row, col])
                    plsc.store_scatter(out_v, [row, col], v + p)
                return carry
            lax.fori_loop(0, _PATCH, kh_body, 0)

    din(0, in0, sem_i0).start()
    din(1, in1, sem_i1).start()

    din(0, in0, sem_i0).wait()
    compute(in0, out0)
    dout(0, out0, sem_o0).start()
    din(2, in0, sem_i0).start()

    din(1, in1, sem_i1).wait()
    compute(in1, out1)
    dout(1, out1, sem_o1).start()
    din(3, in1, sem_i1).start()

    def steady(i, carry):
        b0 = 2 * i
        din(0, in0, sem_i0).wait()
        dout(0, out0, sem_o0).wait()
        compute(in0, out0)
        dout(b0, out0, sem_o0).start()

        @pl.when(b0 + 2 < nb)
        def _():
            din(b0 + 2, in0, sem_i0).start()

        b1 = 2 * i + 1
        din(0, in1, sem_i1).wait()
        dout(0, out1, sem_o1).wait()
        compute(in1, out1)
        dout(b1, out1, sem_o1).start()

        @pl.when(b1 + 2 < nb)
        def _():
            din(b1 + 2, in1, sem_i1).start()

        return carry

    lax.fori_loop(1, nb // 2, steady, 0)

    dout(0, out0, sem_o0).wait()
    dout(0, out1, sem_o1).wait()


def kernel(x, position_embeddings):
    b = x.shape[0]
    table = position_embeddings.reshape(_GRID * _GRID, _E)
    idx = jnp.asarray(_hash_rows())

    mesh = plsc.VectorSubcoreMesh(core_axis_name="c", subcore_axis_name="s")
    run = functools.partial(
        pl.kernel,
        out_type=jax.ShapeDtypeStruct((b, _H2 * _W2, _E), x.dtype),
        mesh=mesh,
        compiler_params=pltpu.CompilerParams(needs_layout_passes=False),
        scratch_types=[
            pltpu.VMEM((3, _PATCH, _WHALF), jnp.float32),
            pltpu.VMEM((3, _PATCH, _WHALF), jnp.float32),
            pltpu.VMEM((8, _E), jnp.float32),
            pltpu.VMEM((8, _E), jnp.float32),
            pltpu.VMEM((8, _E), jnp.float32),
            pltpu.VMEM((8,), jnp.int32),
            pltpu.SemaphoreType.DMA,
            pltpu.SemaphoreType.DMA,
            pltpu.SemaphoreType.DMA,
            pltpu.SemaphoreType.DMA,
            pltpu.SemaphoreType.DMA,
        ],
    )(_sc_body)
    return run(x, table, idx)


# speedup vs baseline: 24.6520x; 2.5841x over previous
import functools

import jax
import jax.numpy as jnp
import numpy as np
from jax import lax
from jax.experimental import pallas as pl
from jax.experimental.pallas import tpu as pltpu
from jax.experimental.pallas import tpu_sc as plsc

_PATCH = 32
_GRID = 10
_H2 = 16
_W2 = 16
_E = _PATCH * _PATCH * 3
_WHALF = 256


def _hash_rows():
    i = np.arange(_H2)
    j = np.arange(_W2)
    hi = np.floor(i.astype(np.float32) * _GRID / _H2).astype(np.int32)
    hj = np.floor(j.astype(np.float32) * _GRID / _W2).astype(np.int32)
    return (hi[:, None] * _GRID + hj[None, :]).reshape(-1)


def _sc_body(x_hbm, tab_hbm, idx_hbm, out_hbm,
             in0, in1, out0, out1, shared, idx8_v,
             sem_i0, sem_i1, sem_o0, sem_o1, sem_p0, sem_p1, sem_t):
    cidx = lax.axis_index("c")
    sidx = lax.axis_index("s")
    wid = sidx * 2 + cidx
    ph = wid // 2
    half = wid % 2
    p0 = ph * _W2 + half * 8
    row0 = ph * _PATCH
    w0 = half * _WHALF
    nb = x_hbm.shape[0]

    def din(b, buf, sem):
        return pltpu.make_async_copy(
            x_hbm.at[b, :, pl.ds(row0, _PATCH), pl.ds(w0, _WHALF)], buf, sem)

    def dout(b, buf, sem):
        return pltpu.make_async_copy(buf, out_hbm.at[b, pl.ds(p0, 8)], sem)

    def prefill(buf, sem):
        return pltpu.make_async_copy(shared.at[sidx], buf, sem)

    din(0, in0, sem_i0).start()
    din(1, in1, sem_i1).start()

    pltpu.sync_copy(idx_hbm.at[pl.ds(p0, 8)], idx8_v)
    pltpu.async_copy(tab_hbm.at[idx8_v], out0, sem_t).wait()
    pltpu.sync_copy(out0, shared.at[sidx])

    prefill(out0, sem_p0).start()
    prefill(out1, sem_p1).start()

    iota3 = lax.iota(jnp.int32, 16) * 3
    rows = [jnp.full((16,), pw, jnp.int32) for pw in range(8)]

    def compute(in_v, out_v):
        for c in range(3):
            @plsc.parallel_loop(0, _PATCH, unroll=2)
            def _(kh, c=c):
                col0 = iota3 + (kh * 96 + c)
                col1 = col0 + 48
                for s in range(16):
                    row = rows[s // 2]
                    col = col0 if s % 2 == 0 else col1
                    v = in_v[c, kh, pl.ds(s * 16, 16)]
                    plsc.addupdate_scatter(out_v, [row, col], v)

    def phase(b, in_v, out_v, sem_i, sem_o, sem_p):
        din(0, in_v, sem_i).wait()
        prefill(out_v, sem_p).wait()
        compute(in_v, out_v)
        dout(b, out_v, sem_o).start()

        @pl.when(b + 2 < nb)
        def _():
            din(b + 2, in_v, sem_i).start()

    phase(0, in0, out0, sem_i0, sem_o0, sem_p0)
    phase(1, in1, out1, sem_i1, sem_o1, sem_p1)

    def steady(i, carry):
        dout(0, out0, sem_o0).wait()
        prefill(out0, sem_p0).start()
        dout(0, out1, sem_o1).wait()
        prefill(out1, sem_p1).start()
        phase(2 * i, in0, out0, sem_i0, sem_o0, sem_p0)
        phase(2 * i + 1, in1, out1, sem_i1, sem_o1, sem_p1)
        return carry

    lax.fori_loop(1, nb // 2, steady, 0)

    dout(0, out0, sem_o0).wait()
    dout(0, out1, sem_o1).wait()


def kernel(x, position_embeddings):
    b = x.shape[0]
    table = position_embeddings.reshape(_GRID * _GRID, _E)
    idx = jnp.asarray(_hash_rows())

    mesh = plsc.VectorSubcoreMesh(core_axis_name="c", subcore_axis_name="s")
    run = functools.partial(
        pl.kernel,
        out_type=jax.ShapeDtypeStruct((b, _H2 * _W2, _E), x.dtype),
        mesh=mesh,
        compiler_params=pltpu.CompilerParams(needs_layout_passes=False),
        scratch_types=[
            pltpu.VMEM((3, _PATCH, _WHALF), jnp.float32),
            pltpu.VMEM((3, _PATCH, _WHALF), jnp.float32),
            pltpu.VMEM((8, _E), jnp.float32),
            pltpu.VMEM((8, _E), jnp.float32),
            pltpu.VMEM_SHARED((16, 8, _E), jnp.float32),
            pltpu.VMEM((8,), jnp.int32),
            pltpu.SemaphoreType.DMA,
            pltpu.SemaphoreType.DMA,
            pltpu.SemaphoreType.DMA,
            pltpu.SemaphoreType.DMA,
            pltpu.SemaphoreType.DMA,
            pltpu.SemaphoreType.DMA,
            pltpu.SemaphoreType.DMA,
        ],
    )(_sc_body)
    return run(x, table, idx)
